# Initial kernel scaffold; baseline (speedup 1.0000x reference)
import jax
import jax.numpy as jnp
from jax.experimental import pallas as pl


def kernel(x_nchw, params):
    raise NotImplementedError("write your pallas kernel here")


# speedup vs baseline: 7.8125x; 7.8125x over previous
import functools

import jax
import jax.numpy as jnp
from jax.experimental import pallas as pl
from jax.experimental.pallas import tpu as pltpu


def _upconv_matmul_kernel(params_ref, x_ref, o_ref, m_ref, *, h, w):
    a = params_ref[0]
    b = params_ref[1]
    c = params_ref[2]
    s_dim = h * w
    d_dim = 4 * h * w

    @pl.when(pl.program_id(1) == 0)
    def _build_m():
        ch = 512 if d_dim % 512 == 0 else d_dim
        for k in range(d_dim // ch):
            s = jax.lax.broadcasted_iota(jnp.int32, (s_dim, ch), 0)
            d = jax.lax.broadcasted_iota(jnp.int32, (s_dim, ch), 1) + k * ch
            si = s // w
            sj = s % w
            m = d // (2 * w)
            n = d % (2 * w)
            i = m // 2
            p = m % 2
            j = n // 2
            q = n % 2
            nb_i = jnp.where(p == 0, (i + h - 1) % h, (i + 1) % h)
            side_j = jnp.where(q == 0, (j + w - 1) % w, (j + 1) % w)
            rc = (si == i).astype(jnp.float32)
            rn = (si == nb_i).astype(jnp.float32)
            cc = (sj == j).astype(jnp.float32)
            cs = (sj == side_j).astype(jnp.float32)
            wa = (a + b) * cc + a * cs
            wb = (b + c) * cc + b * cs
            m_ref[:, k * ch:(k + 1) * ch] = (rc * (wa + wb) + rn * wa
                                             ).astype(jnp.bfloat16)

    bias = -(4.0 * a + 4.0 * b + c) * 0.5
    xb = x_ref[...].astype(jnp.bfloat16)
    o_ref[...] = jnp.dot(xb, m_ref[...],
                         preferred_element_type=jnp.float32) + bias


def kernel(x_nchw, params):
    n, ch, h, w = x_nchw.shape
    assert ch == 1
    s_dim = h * w
    d_dim = 4 * h * w

    params = params.astype(jnp.float32)
    x = x_nchw.astype(jnp.float32).reshape(n, s_dim)

    bm = min(512, n)
    g0 = 2
    n_pad = ((n + bm * g0 - 1) // (bm * g0)) * (bm * g0)
    if n_pad != n:
        x = jnp.pad(x, ((0, n_pad - n), (0, 0)))
    g1 = n_pad // (bm * g0)

    out = pl.pallas_call(
        functools.partial(_upconv_matmul_kernel, h=h, w=w),
        out_shape=jax.ShapeDtypeStruct((n_pad, d_dim), jnp.float32),
        grid_spec=pltpu.PrefetchScalarGridSpec(
            num_scalar_prefetch=1,
            grid=(g0, g1),
            in_specs=[pl.BlockSpec((bm, s_dim), lambda i, j, p: (i * g1 + j, 0))],
            out_specs=pl.BlockSpec((bm, d_dim), lambda i, j, p: (i * g1 + j, 0)),
            scratch_shapes=[pltpu.VMEM((s_dim, d_dim), jnp.bfloat16)],
        ),
        compiler_params=pltpu.CompilerParams(
            dimension_semantics=("parallel", "arbitrary"),
            vmem_limit_bytes=56 * 1024 * 1024,
        ),
    )(params, x)

    return out[:n].reshape(n, 2 * h, 2 * w)[:, None]
